# baseline (speedup 1.0000x reference)
import functools

import jax
import jax.numpy as jnp
from jax import lax
from jax.experimental import pallas as pl
from jax.experimental.pallas import tpu as pltpu
from jax.experimental.pallas import tpu_sc as plsc

L_EVEN = 1664
L_ODD = 1536


@functools.lru_cache(maxsize=None)
def _build_gather(n_idx: int, dim: int):
    info = plsc.get_sparse_core_info()
    nw = info.num_cores * info.num_subcores
    assert n_idx % nw == 0
    per_w = n_idx // nw
    assert per_w * 2 == L_EVEN + L_ODD
    mesh = plsc.VectorSubcoreMesh(core_axis_name="c", subcore_axis_name="s")

    @functools.partial(
        pl.kernel,
        mesh=mesh,
        out_type=jax.ShapeDtypeStruct((dim, n_idx), jnp.float32),
        compiler_params=pltpu.CompilerParams(use_tc_tiling_on_sc=False),
        scratch_types=[pltpu.VMEM((L_EVEN,), jnp.int32)]
        + [pltpu.VMEM((L_EVEN,), jnp.float32) for _ in range(dim)]
        + [pltpu.SemaphoreType.DMA, pltpu.SemaphoreType.DMA],
    )
    def gather(tableT, idx_hbm, out_hbm, idx_v, *rest):
        cb = rest[:dim]
        gsem, osem = rest[dim], rest[dim + 1]
        wid = lax.axis_index("s") * info.num_cores + lax.axis_index("c")
        start = wid * per_w + (wid % 2) * (L_EVEN - per_w)

        def run(length):
            sl = pl.ds(0, length)
            pltpu.sync_copy(idx_hbm.at[pl.ds(start, length)], idx_v.at[sl])
            gh = [
                pltpu.async_copy(tableT.at[c].at[idx_v.at[sl]], cb[c].at[sl], gsem)
                for c in range(dim)
            ]
            oh = []
            for c in range(dim):
                gh[c].wait()
                oh.append(pltpu.async_copy(
                    cb[c].at[sl], out_hbm.at[c, pl.ds(start, length)], osem))
            for h in oh:
                h.wait()

        @pl.when(wid % 2 == 0)
        def _():
            run(L_EVEN)

        @pl.when(wid % 2 == 1)
        def _():
            run(L_ODD)

    return gather


def kernel(kwids, kw_dist_adj, mask, word_embed_table):
    vocab, dim = word_embed_table.shape
    idx = kwids.reshape(-1)
    gather = _build_gather(idx.shape[0], dim)
    rowsT = gather(word_embed_table.T, idx)
    kw_embed = rowsT.T.reshape(kwids.shape + (dim,))
    return (kw_embed, kw_dist_adj, mask)
